# baseline (speedup 1.0000x reference)
import functools

import jax
import jax.numpy as jnp
from jax import lax
from jax.experimental import pallas as pl
from jax.experimental.pallas import tpu as pltpu
from jax.experimental.pallas import tpu_sc as plsc

_NUM_CORES = 2
_NUM_SUBCORES = 16
_NUM_WORKERS = _NUM_CORES * _NUM_SUBCORES


@functools.lru_cache(maxsize=None)
def _build_copy_kernel(B, L, D, dtype):
    rows_per_w = L // _NUM_WORKERS
    mesh = plsc.VectorSubcoreMesh(core_axis_name="c", subcore_axis_name="s")

    @functools.partial(
        pl.kernel,
        mesh=mesh,
        out_type=jax.ShapeDtypeStruct((B * L, D), dtype),
        scratch_types=[
            pltpu.SemaphoreType.DMA,
        ],
    )
    def copy_kernel(w_hbm, out_hbm, sem):
        wid = lax.axis_index("s") * _NUM_CORES + lax.axis_index("c")
        base = wid * rows_per_w
        copies = [
            pltpu.async_copy(
                w_hbm.at[pl.ds(base, rows_per_w)],
                out_hbm.at[pl.ds(b * L + base, rows_per_w)],
                sem,
            )
            for b in range(B)
        ]
        for cp in copies:
            cp.wait()

    return copy_kernel


def kernel(x, W):
    B, L, D = x.shape
    out_flat = _build_copy_kernel(B, L, D, W.dtype)(W[:L])
    return out_flat.reshape(B, L, D)


# speedup vs baseline: 44.9371x; 44.9371x over previous
import functools

import jax
import jax.numpy as jnp
from jax import lax
from jax.experimental import pallas as pl
from jax.experimental.pallas import tpu as pltpu
from jax.experimental.pallas import tpu_sc as plsc

_NUM_CORES = 2
_NUM_SUBCORES = 16
_NUM_WORKERS = _NUM_CORES * _NUM_SUBCORES
_MAX_CHUNK = 48
_NBUF = 2
_PRIME = 1


def _chunk_sizes(total):
    sizes = []
    left = total
    while left > 0:
        take = min(_MAX_CHUNK, left)
        sizes.append(take)
        left -= take
    return sizes


@functools.lru_cache(maxsize=None)
def _build_copy_kernel(B, L, D, dtype):
    rows_per_w = L // _NUM_WORKERS
    sizes = _chunk_sizes(rows_per_w)
    offs = [sum(sizes[:i]) for i in range(len(sizes))]
    n_chunks = len(sizes)
    mesh = plsc.VectorSubcoreMesh(core_axis_name="c", subcore_axis_name="s")

    @functools.partial(
        pl.kernel,
        mesh=mesh,
        out_type=jax.ShapeDtypeStruct((B * L, D), dtype),
        scratch_types=[
            pltpu.VMEM((_NBUF, _MAX_CHUNK, D), dtype),
            pltpu.SemaphoreType.DMA,
            pltpu.SemaphoreType.DMA,
        ],
    )
    def copy_kernel(w_hbm, out_hbm, buf, in_sem, out_sem):
        wid = lax.axis_index("s") * _NUM_CORES + lax.axis_index("c")
        base = wid * rows_per_w

        def start_read(c):
            return pltpu.async_copy(
                w_hbm.at[pl.ds(base + offs[c], sizes[c])],
                buf.at[c % _NBUF, pl.ds(0, sizes[c])],
                in_sem,
            )

        def start_writes(c):
            return [
                pltpu.async_copy(
                    buf.at[c % _NBUF, pl.ds(0, sizes[c])],
                    out_hbm.at[pl.ds(b * L + base + offs[c], sizes[c])],
                    out_sem,
                )
                for b in range(B)
            ]

        prime = min(_PRIME, n_chunks)
        reads = {c: start_read(c) for c in range(prime)}
        pending_writes = {}
        for c in range(n_chunks):
            reads.pop(c).wait()
            pending_writes[c] = start_writes(c)
            nxt = c + prime
            if nxt < n_chunks:
                old = nxt - _NBUF
                if old in pending_writes:
                    for w in pending_writes.pop(old):
                        w.wait()
                reads[nxt] = start_read(nxt)
        for group in pending_writes.values():
            for w in group:
                w.wait()

    return copy_kernel


def kernel(x, W):
    B, L, D = x.shape
    out_flat = _build_copy_kernel(B, L, D, W.dtype)(W[:L])
    return out_flat.reshape(B, L, D)
